# baseline (speedup 1.0000x reference)
import functools
import math

import jax
import jax.numpy as jnp
from jax import lax
from jax.experimental import pallas as pl
from jax.experimental.pallas import tpu as pltpu
from jax.experimental.pallas import tpu_sc as plsc

_NUM_USERS = 100000
_POOL_SIZE = 200
_NUM_NEG = 20
_BATCH = 16384

_NC = 2
_NS = 16
_L = 16
_NW = _NC * _NS
_BPW = _BATCH // _NW
_EPW = _BPW * _NUM_NEG
_GCHUNK = 128
_NGC = _BPW // _GCHUNK


def _tec_body(user_hbm, pool_hbm, idxk_hbm, out_hbm,
              user_v, rows_v, idx_v, out_v, sem):
    wid = lax.axis_index("s") * _NC + lax.axis_index("c")
    base = wid * _BPW
    ebase = base * _NUM_NEG

    pltpu.sync_copy(user_hbm.at[pl.ds(base, _BPW)], user_v)
    pltpu.sync_copy(idxk_hbm.at[pl.ds(ebase, _EPW)], idx_v)

    copies = []
    for j in range(_NGC):
        copies.append(pltpu.async_copy(
            pool_hbm.at[user_v.at[pl.ds(j * _GCHUNK, _GCHUNK)]],
            rows_v.at[pl.ds(j * _GCHUNK, _GCHUNK)],
            sem,
        ))
    for c in copies:
        c.wait()

    iota = lax.iota(jnp.int32, _L)

    def body(e, carry):
        o = e * _L
        lanes = o + iota
        b_loc = lax.div(lanes, jnp.int32(_NUM_NEG))
        col = idx_v[pl.ds(o, _L)]
        vals = plsc.load_gather(rows_v, [b_loc, col])
        out_v[pl.ds(o, _L)] = vals
        return carry

    lax.fori_loop(0, _EPW // _L, body, 0)

    pltpu.sync_copy(out_v, out_hbm.at[pl.ds(ebase, _EPW)])


def kernel(user_id, pool, idx_k):
    mesh = plsc.VectorSubcoreMesh(core_axis_name="c", subcore_axis_name="s")
    kfn = pl.kernel(
        _tec_body,
        mesh=mesh,
        compiler_params=pltpu.CompilerParams(
            use_tc_tiling_on_sc=False, needs_layout_passes=False),
        out_type=jax.ShapeDtypeStruct((_BATCH * _NUM_NEG,), jnp.int32),
        scratch_types=[
            pltpu.VMEM((_BPW,), jnp.int32),
            pltpu.VMEM((_BPW, _POOL_SIZE), jnp.int32),
            pltpu.VMEM((_EPW,), jnp.int32),
            pltpu.VMEM((_EPW,), jnp.int32),
            pltpu.SemaphoreType.DMA,
        ],
    )
    neg_flat = kfn(user_id, pool, idx_k.reshape(-1))
    neg_items = neg_flat.reshape(_BATCH, _NUM_NEG)
    log_q = jnp.full((_BATCH, _NUM_NEG), -math.log(float(_POOL_SIZE)),
                     dtype=jnp.float32)
    return neg_items, log_q
